# baseline (speedup 1.0000x reference)
import numpy as np
import jax
import jax.numpy as jnp
from jax import lax
from jax.experimental import pallas as pl
from jax.experimental.pallas import tpu as pltpu
from jax.experimental.pallas import tpu_sc as plsc

_N = 5000
_NPAD = 5120
_NCH = 320
_K = 100
_NEG = -1e30
_CLIP = float(np.float32(np.log(1333.0 / 16.0)))


def _sc_body(inp_hbm, out_hbm, chan_v, x1_v, y1_v, x2_v, y2_v, ar_v, wk_v,
             cs_v, sn_v, out_v):
    cid = lax.axis_index("c")
    sid = lax.axis_index("s")

    @pl.when((cid == 0) & (sid == 0))
    def _():
        pltpu.sync_copy(inp_hbm, chan_v)
        lane = lax.iota(jnp.int32, 16)
        lanef = lane.astype(jnp.float32)
        zv = jnp.zeros((16,), jnp.float32)

        def dec(j, _):
            r = j // 8
            sl = pl.ds((j % 8) * 16, 16)
            px1 = chan_v[0, r, sl]
            py1 = chan_v[1, r, sl]
            px2 = chan_v[2, r, sl]
            py2 = chan_v[3, r, sl]
            tx = chan_v[4, r, sl] / 10.0
            ty = chan_v[5, r, sl] / 10.0
            tw = chan_v[6, r, sl] / 5.0
            th = chan_v[7, r, sl] / 5.0
            score = chan_v[8, r, sl]
            wa = px2 - px1
            ha = py2 - py1
            xa = (px2 + px1) * 0.5
            ya = (py2 + py1) * 0.5
            wb = jnp.exp(jnp.minimum(tw, _CLIP)) * wa
            hb = jnp.exp(jnp.minimum(th, _CLIP)) * ha
            xb = tx * wa + xa
            yb = ty * ha + ya
            x1 = jnp.clip(xb - wb * 0.5, 0.0, 1024.0)
            y1 = jnp.clip(yb - hb * 0.5, 0.0, 1024.0)
            x2 = jnp.clip(xb + wb * 0.5, 0.0, 1024.0)
            y2 = jnp.clip(yb + hb * 0.5, 0.0, 1024.0)
            area = jnp.maximum(x2 - x1, 0.0) * jnp.maximum(y2 - y1, 0.0)
            gidx = j * 16 + lane
            valid = (score > 0.05) & ((x2 - x1) * (y2 - y1) > 0.0) & (gidx < _N)
            x1_v[r, sl] = x1
            y1_v[r, sl] = y1
            x2_v[r, sl] = x2
            y2_v[r, sl] = y2
            ar_v[r, sl] = area
            wk_v[r, sl] = jnp.where(valid, score, _NEG)
            cs_v[r, sl] = chan_v[9, r, sl]
            sn_v[r, sl] = chan_v[10, r, sl]
            return 0

        lax.fori_loop(0, _NCH, dec, 0)

        def fold(a, b):
            take = (b[0] > a[0]) | ((b[0] == a[0]) & (b[1] < a[1]))
            return tuple(jnp.where(take, q, p) for p, q in zip(a, b))

        neg_carry = (jnp.full((16,), _NEG, jnp.float32),
                     jnp.full((16,), 1e9, jnp.float32),
                     zv, zv, zv, zv, zv, zv)

        def supp(bx1, by1, bx2, by2, bar):
            def body(j, carry):
                r = j // 8
                sl = pl.ds((j % 8) * 16, 16)
                cx1 = x1_v[r, sl]
                cy1 = y1_v[r, sl]
                cx2 = x2_v[r, sl]
                cy2 = y2_v[r, sl]
                xx1 = jnp.maximum(cx1, bx1)
                yy1 = jnp.maximum(cy1, by1)
                xx2 = jnp.minimum(cx2, bx2)
                yy2 = jnp.minimum(cy2, by2)
                inter = (jnp.maximum(xx2 - xx1, 0.0)
                         * jnp.maximum(yy2 - yy1, 0.0))
                union = ar_v[r, sl] + bar - inter
                iou = inter / jnp.maximum(union, 1e-8)
                w = jnp.where(iou >= 0.5, _NEG, wk_v[r, sl])
                wk_v[r, sl] = w
                cand = (w, j * 16.0 + lanef, cx1, cy1, cx2, cy2,
                        cs_v[r, sl], sn_v[r, sl])
                return fold(carry, cand)

            return lax.fori_loop(0, _NCH, body, neg_carry)

        def permute(v, perm):
            return lax.gather(
                v, perm[:, None],
                lax.GatherDimensionNumbers(offset_dims=(),
                                           collapsed_slice_dims=(0,),
                                           start_index_map=(0,)),
                (1,), mode=lax.GatherScatterMode.PROMISE_IN_BOUNDS)

        def lanefold(carry):
            for sh in (8, 4, 2, 1):
                perm = (lane + sh) % 16
                carry = fold(carry, tuple(permute(c, perm) for c in carry))
            return carry

        def nms(i, carry):
            m, _fi, bx1, by1, bx2, by2, bco, bsi = lanefold(carry)
            vf = jnp.where(m > _NEG * 0.5, 1.0, 0.0)
            vals = jnp.where(
                lane == 0, bx1,
                jnp.where(lane == 1, by1,
                          jnp.where(lane == 2, bx2,
                                    jnp.where(lane == 3, by2,
                                              jnp.where(lane == 4, 1.0,
                                                        jnp.where(lane == 5, m,
                                                                  jnp.where(lane == 6, bco, bsi))))))) * vf
            out_v[i, :] = vals
            bar = jnp.maximum(bx2 - bx1, 0.0) * jnp.maximum(by2 - by1, 0.0)
            return supp(bx1, by1, bx2, by2, bar)

        carry0 = supp(zv, zv, zv, zv, zv)
        lax.fori_loop(0, _K, nms, carry0)
        pltpu.sync_copy(out_v, out_hbm)


def kernel(proposals, box_logits, label_logits, box_cos_logits, box_sin_logits):
    x = jnp.concatenate([proposals, box_logits, label_logits[:, 1:2],
                         box_cos_logits[:, None], box_sin_logits[:, None]],
                        axis=1)
    x = jnp.pad(x, ((0, _NPAD - _N), (0, 0)))
    inp = x.T.reshape(11, 40, 128)
    mesh = plsc.VectorSubcoreMesh(core_axis_name="c", subcore_axis_name="s")
    f = pl.kernel(
        _sc_body, mesh=mesh,
        out_type=jax.ShapeDtypeStruct((_K, 16), jnp.float32),
        scratch_types=[
            pltpu.VMEM((11, 40, 128), jnp.float32),
            pltpu.VMEM((40, 128), jnp.float32),
            pltpu.VMEM((40, 128), jnp.float32),
            pltpu.VMEM((40, 128), jnp.float32),
            pltpu.VMEM((40, 128), jnp.float32),
            pltpu.VMEM((40, 128), jnp.float32),
            pltpu.VMEM((40, 128), jnp.float32),
            pltpu.VMEM((40, 128), jnp.float32),
            pltpu.VMEM((40, 128), jnp.float32),
            pltpu.VMEM((_K, 16), jnp.float32),
        ],
    )
    out = f(inp)
    return out[:, :8]


# speedup vs baseline: 7.5023x; 7.5023x over previous
import numpy as np
import jax
import jax.numpy as jnp
from jax.experimental import pallas as pl
from jax.experimental.pallas import tpu as pltpu

_N = 5000
_ROWS = 40
_NPAD = _ROWS * 128
_K = 100
_NEG = -1e30
_CLIP = float(np.float32(np.log(1333.0 / 16.0)))


def _nms_body(inp_ref, out_ref):
    px1 = inp_ref[0]
    py1 = inp_ref[1]
    px2 = inp_ref[2]
    py2 = inp_ref[3]
    tx = inp_ref[4] / 10.0
    ty = inp_ref[5] / 10.0
    tw = inp_ref[6] / 5.0
    th = inp_ref[7] / 5.0
    score = inp_ref[8]
    cosl = inp_ref[9]
    sinl = inp_ref[10]

    wa = px2 - px1
    ha = py2 - py1
    xa = (px2 + px1) * 0.5
    ya = (py2 + py1) * 0.5
    wb = jnp.exp(jnp.minimum(tw, _CLIP)) * wa
    hb = jnp.exp(jnp.minimum(th, _CLIP)) * ha
    xb = tx * wa + xa
    yb = ty * ha + ya
    x1 = jnp.clip(xb - wb * 0.5, 0.0, 1024.0)
    y1 = jnp.clip(yb - hb * 0.5, 0.0, 1024.0)
    x2 = jnp.clip(xb + wb * 0.5, 0.0, 1024.0)
    y2 = jnp.clip(yb + hb * 0.5, 0.0, 1024.0)

    area = jnp.maximum(x2 - x1, 0.0) * jnp.maximum(y2 - y1, 0.0)
    idx = (jax.lax.broadcasted_iota(jnp.int32, (_ROWS, 128), 0) * 128
           + jax.lax.broadcasted_iota(jnp.int32, (_ROWS, 128), 1))
    valid0 = (score > 0.05) & ((x2 - x1) * (y2 - y1) > 0.0) & (idx < _N)
    work0 = jnp.where(valid0, score, _NEG)

    sub8 = jax.lax.broadcasted_iota(jnp.int32, (8, 128), 0)
    lane8 = jax.lax.broadcasted_iota(jnp.int32, (8, 128), 1)
    acc0 = jnp.zeros((8, 128), jnp.float32)
    fidx = idx.astype(jnp.float32)

    z = jnp.float32(0.0)

    def body(i, state):
        work, acc = state

        bval = jnp.max(work)
        sel0 = work == bval

        def gather(sel):
            return [jnp.sum(jnp.where(sel, c, z))
                    for c in (x1, y1, x2, y2, cosl, sinl)] + [
                        jnp.sum(jnp.where(sel, 1.0, z))]

        def step(sel, work_in, acc_in):
            bx1, by1, bx2, by2, bco, bsi, cnt = gather(sel)
            bar = jnp.maximum(bx2 - bx1, 0.0) * jnp.maximum(by2 - by1, 0.0)
            xx1 = jnp.maximum(x1, bx1)
            yy1 = jnp.maximum(y1, by1)
            xx2 = jnp.minimum(x2, bx2)
            yy2 = jnp.minimum(y2, by2)
            inter = jnp.maximum(xx2 - xx1, 0.0) * jnp.maximum(yy2 - yy1, 0.0)
            union = area + bar - inter
            iou = inter / jnp.maximum(union, 1e-8)
            work_out = jnp.where(iou >= 0.5, _NEG, work_in)
            vf = (bval > _NEG * 0.5).astype(jnp.float32)
            vals = jnp.where(
                sub8 == 0, bx1,
                jnp.where(sub8 == 1, by1,
                          jnp.where(sub8 == 2, bx2,
                                    jnp.where(sub8 == 3, by2,
                                              jnp.where(sub8 == 4, 1.0,
                                                        jnp.where(sub8 == 5, bval,
                                                                  jnp.where(sub8 == 6, bco, bsi))))))) * vf
            acc_out = jnp.where(lane8 == i, vals, acc_in)
            return work_out, acc_out, cnt

        work1, acc1, cnt = step(sel0, work, acc)

        def tie_fix(_):
            bfi = jnp.min(jnp.where(sel0, fidx, jnp.float32(2 ** 30)))
            w2, a2, _unused = step(fidx == bfi, work, acc)
            return w2, a2

        work2, acc2 = jax.lax.cond(
            cnt > 1.5, tie_fix, lambda _: (work1, acc1), None)
        return work2, acc2

    _, acc = jax.lax.fori_loop(0, _K, body, (work0, acc0))
    out_ref[...] = acc


def kernel(proposals, box_logits, label_logits, box_cos_logits, box_sin_logits):
    x = jnp.concatenate([proposals, box_logits, label_logits[:, 1:2],
                         box_cos_logits[:, None], box_sin_logits[:, None]],
                        axis=1)
    x = jnp.pad(x, ((0, _NPAD - _N), (0, 0)))
    inp = x.T.reshape(11, _ROWS, 128)
    out = pl.pallas_call(
        _nms_body,
        out_shape=jax.ShapeDtypeStruct((8, 128), jnp.float32),
    )(inp)
    return out[:, :_K].T


# speedup vs baseline: 7.5029x; 1.0001x over previous
import numpy as np
import jax
import jax.numpy as jnp
from jax.experimental import pallas as pl
from jax.experimental.pallas import tpu as pltpu

_N = 5000
_ROWS = 40
_NPAD = _ROWS * 128
_K = 100
_NEG = -1e30
_CLIP = float(np.float32(np.log(1333.0 / 16.0)))


def _nms_body(inp_ref, out_ref):
    px1 = inp_ref[0]
    py1 = inp_ref[1]
    px2 = inp_ref[2]
    py2 = inp_ref[3]
    tx = inp_ref[4] / 10.0
    ty = inp_ref[5] / 10.0
    tw = inp_ref[6] / 5.0
    th = inp_ref[7] / 5.0
    score = inp_ref[8]
    cosl = inp_ref[9]
    sinl = inp_ref[10]

    wa = px2 - px1
    ha = py2 - py1
    xa = (px2 + px1) * 0.5
    ya = (py2 + py1) * 0.5
    wb = jnp.exp(jnp.minimum(tw, _CLIP)) * wa
    hb = jnp.exp(jnp.minimum(th, _CLIP)) * ha
    xb = tx * wa + xa
    yb = ty * ha + ya
    x1 = jnp.clip(xb - wb * 0.5, 0.0, 1024.0)
    y1 = jnp.clip(yb - hb * 0.5, 0.0, 1024.0)
    x2 = jnp.clip(xb + wb * 0.5, 0.0, 1024.0)
    y2 = jnp.clip(yb + hb * 0.5, 0.0, 1024.0)

    idx = (jax.lax.broadcasted_iota(jnp.int32, (_ROWS, 128), 0) * 128
           + jax.lax.broadcasted_iota(jnp.int32, (_ROWS, 128), 1))
    valid0 = (score > 0.05) & ((x2 - x1) * (y2 - y1) > 0.0) & (idx < _N)
    work0 = jnp.where(valid0, score, _NEG)

    sub8 = jax.lax.broadcasted_iota(jnp.int32, (8, 128), 0)
    lane8 = jax.lax.broadcasted_iota(jnp.int32, (8, 128), 1)
    acc0 = jnp.zeros((8, 128), jnp.float32)

    z = jnp.float32(0.0)

    def body(i, state):
        work, acc = state

        bval = jnp.max(work)
        sel0 = work == bval

        def gather(sel):
            return [jnp.sum(jnp.where(sel, c, z))
                    for c in (x1, y1, x2, y2, inp_ref[9], inp_ref[10])] + [
                        jnp.sum(jnp.where(sel, 1.0, z))]

        def step(sel, work_in, acc_in):
            bx1, by1, bx2, by2, bco, bsi, cnt = gather(sel)
            bar = jnp.maximum(bx2 - bx1, 0.0) * jnp.maximum(by2 - by1, 0.0)
            xx1 = jnp.maximum(x1, bx1)
            yy1 = jnp.maximum(y1, by1)
            xx2 = jnp.minimum(x2, bx2)
            yy2 = jnp.minimum(y2, by2)
            inter = jnp.maximum(xx2 - xx1, 0.0) * jnp.maximum(yy2 - yy1, 0.0)
            area = jnp.maximum(x2 - x1, 0.0) * jnp.maximum(y2 - y1, 0.0)
            union = area + bar - inter
            iou = inter / jnp.maximum(union, 1e-8)
            work_out = jnp.where(iou >= 0.5, _NEG, work_in)
            vf = (bval > _NEG * 0.5).astype(jnp.float32)
            vals = jnp.where(
                sub8 == 0, bx1,
                jnp.where(sub8 == 1, by1,
                          jnp.where(sub8 == 2, bx2,
                                    jnp.where(sub8 == 3, by2,
                                              jnp.where(sub8 == 4, 1.0,
                                                        jnp.where(sub8 == 5, bval,
                                                                  jnp.where(sub8 == 6, bco, bsi))))))) * vf
            acc_out = jnp.where(lane8 == i, vals, acc_in)
            return work_out, acc_out, cnt

        work1, acc1, cnt = step(sel0, work, acc)

        def tie_fix(_):
            fidx = (jax.lax.broadcasted_iota(jnp.int32, (_ROWS, 128), 0) * 128
                    + jax.lax.broadcasted_iota(jnp.int32, (_ROWS, 128), 1)
                    ).astype(jnp.float32)
            bfi = jnp.min(jnp.where(sel0, fidx, jnp.float32(2 ** 30)))
            w2, a2, _unused = step(fidx == bfi, work, acc)
            return w2, a2

        work2, acc2 = jax.lax.cond(
            cnt > 1.5, tie_fix, lambda _: (work1, acc1), None)
        return work2, acc2

    _, acc = jax.lax.fori_loop(0, _K, body, (work0, acc0))
    out_ref[...] = acc


def kernel(proposals, box_logits, label_logits, box_cos_logits, box_sin_logits):
    x = jnp.concatenate([proposals, box_logits, label_logits[:, 1:2],
                         box_cos_logits[:, None], box_sin_logits[:, None]],
                        axis=1)
    x = jnp.pad(x, ((0, _NPAD - _N), (0, 0)))
    inp = x.T.reshape(11, _ROWS, 128)
    out = pl.pallas_call(
        _nms_body,
        out_shape=jax.ShapeDtypeStruct((8, 128), jnp.float32),
    )(inp)
    return out[:, :_K].T


# speedup vs baseline: 8.7445x; 1.1655x over previous
import numpy as np
import jax
import jax.numpy as jnp
from jax.experimental import pallas as pl

_N = 5000
_ROWS = 40
_NPAD = _ROWS * 128
_K = 100
_NEG = -1e30
_CLIP = float(np.float32(np.log(1333.0 / 16.0)))


def _nms_body(inp_ref, out_ref):
    px1 = inp_ref[0]
    py1 = inp_ref[1]
    px2 = inp_ref[2]
    py2 = inp_ref[3]
    tx = inp_ref[4] / 10.0
    ty = inp_ref[5] / 10.0
    tw = inp_ref[6] / 5.0
    th = inp_ref[7] / 5.0
    score = inp_ref[8]

    wa = px2 - px1
    ha = py2 - py1
    xa = (px2 + px1) * 0.5
    ya = (py2 + py1) * 0.5
    wb = jnp.exp(jnp.minimum(tw, _CLIP)) * wa
    hb = jnp.exp(jnp.minimum(th, _CLIP)) * ha
    xb = tx * wa + xa
    yb = ty * ha + ya
    x1 = jnp.clip(xb - wb * 0.5, 0.0, 1024.0)
    y1 = jnp.clip(yb - hb * 0.5, 0.0, 1024.0)
    x2 = jnp.clip(xb + wb * 0.5, 0.0, 1024.0)
    y2 = jnp.clip(yb + hb * 0.5, 0.0, 1024.0)

    idx = (jax.lax.broadcasted_iota(jnp.int32, (_ROWS, 128), 0) * 128
           + jax.lax.broadcasted_iota(jnp.int32, (_ROWS, 128), 1))
    valid0 = (score > 0.05) & ((x2 - x1) * (y2 - y1) > 0.0) & (idx < _N)
    work0 = jnp.where(valid0, score, _NEG)

    sub8 = jax.lax.broadcasted_iota(jnp.int32, (8, 128), 0)
    lane8 = jax.lax.broadcasted_iota(jnp.int32, (8, 128), 1)
    acc0 = jnp.zeros((8, 128), jnp.float32)
    z = jnp.float32(0.0)

    def gather(sel):
        return [jnp.sum(jnp.where(sel, c, z))
                for c in (x1, y1, x2, y2, inp_ref[9], inp_ref[10])] + [
                    jnp.sum(jnp.where(sel, 1.0, z))]

    def iou_row(bx1, by1, bx2, by2, bar):
        xx1 = jnp.maximum(x1, bx1)
        yy1 = jnp.maximum(y1, by1)
        xx2 = jnp.minimum(x2, bx2)
        yy2 = jnp.minimum(y2, by2)
        inter = jnp.maximum(xx2 - xx1, 0.0) * jnp.maximum(yy2 - yy1, 0.0)
        area = jnp.maximum(x2 - x1, 0.0) * jnp.maximum(y2 - y1, 0.0)
        union = area + bar - inter
        return inter / jnp.maximum(union, 1e-8)

    def row_vals(bx1, by1, bx2, by2, bco, bsi, m):
        return jnp.where(
            sub8 == 0, bx1,
            jnp.where(sub8 == 1, by1,
                      jnp.where(sub8 == 2, bx2,
                                jnp.where(sub8 == 3, by2,
                                          jnp.where(sub8 == 4, 1.0,
                                                    jnp.where(sub8 == 5, m,
                                                              jnp.where(sub8 == 6, bco, bsi)))))))

    def cond_fun(state):
        _work, _acc, cnt, alive = state
        return (cnt < _K) & alive

    def body_fun(state):
        work, acc, cnt, _alive = state
        bval = jnp.max(work)
        sel1 = work == bval
        bx1, by1, bx2, by2, bco, bsi, cnt1 = gather(sel1)
        bval2 = jnp.max(jnp.where(sel1, _NEG, work))
        v1 = (bval > _NEG * 0.5).astype(jnp.float32)

        def normal(_):
            bar1 = jnp.maximum(bx2 - bx1, 0.0) * jnp.maximum(by2 - by1, 0.0)
            iou1 = iou_row(bx1, by1, bx2, by2, bar1)
            w1 = jnp.where(iou1 >= 0.5, _NEG, work)
            acc1 = jnp.where(lane8 == cnt,
                             row_vals(bx1, by1, bx2, by2, bco, bsi, bval) * v1,
                             acc)
            sel2 = work == bval2
            cx1, cy1, cx2, cy2, cco, csi, cnt2 = gather(sel2)
            bar2 = jnp.maximum(cx2 - cx1, 0.0) * jnp.maximum(cy2 - cy1, 0.0)
            sxx1 = jnp.maximum(cx1, bx1)
            syy1 = jnp.maximum(cy1, by1)
            sxx2 = jnp.minimum(cx2, bx2)
            syy2 = jnp.minimum(cy2, by2)
            inter12 = (jnp.maximum(sxx2 - sxx1, 0.0)
                       * jnp.maximum(syy2 - syy1, 0.0))
            iou12 = inter12 / jnp.maximum(bar2 + bar1 - inter12, 1e-8)
            keep2 = (bval2 > _NEG * 0.5) & (iou12 < 0.5) & (cnt2 < 1.5)
            iou2 = iou_row(cx1, cy1, cx2, cy2, bar2)
            w2 = jnp.where(keep2 & (iou2 >= 0.5), _NEG, w1)
            acc2 = jnp.where((lane8 == cnt + 1) & keep2,
                             row_vals(cx1, cy1, cx2, cy2, cco, csi, bval2),
                             acc1)
            return w2, acc2, cnt + 1 + keep2.astype(jnp.int32)

        def tie(_):
            fidx = (jax.lax.broadcasted_iota(jnp.int32, (_ROWS, 128), 0) * 128
                    + jax.lax.broadcasted_iota(jnp.int32, (_ROWS, 128), 1)
                    ).astype(jnp.float32)
            bfi = jnp.min(jnp.where(sel1, fidx, jnp.float32(2 ** 30)))
            selt = fidx == bfi
            tx1, ty1, tx2, ty2, tco, tsi, _c = gather(selt)
            bart = jnp.maximum(tx2 - tx1, 0.0) * jnp.maximum(ty2 - ty1, 0.0)
            iout = iou_row(tx1, ty1, tx2, ty2, bart)
            wt = jnp.where(iout >= 0.5, _NEG, work)
            acct = jnp.where(lane8 == cnt,
                             row_vals(tx1, ty1, tx2, ty2, tco, tsi, bval) * v1,
                             acc)
            return wt, acct, cnt + 1

        work_n, acc_n, cnt_n = jax.lax.cond(cnt1 > 1.5, tie, normal, None)
        alive = bval > _NEG * 0.5
        cnt_n = jnp.where(alive, cnt_n, cnt)
        return work_n, acc_n, cnt_n, alive

    state = jax.lax.while_loop(
        cond_fun, body_fun, (work0, acc0, jnp.int32(0), jnp.bool_(True)))
    out_ref[...] = state[1]


def kernel(proposals, box_logits, label_logits, box_cos_logits, box_sin_logits):
    x = jnp.concatenate([proposals, box_logits, label_logits[:, 1:2],
                         box_cos_logits[:, None], box_sin_logits[:, None]],
                        axis=1)
    x = jnp.pad(x, ((0, _NPAD - _N), (0, 0)))
    inp = x.T.reshape(11, _ROWS, 128)
    out = pl.pallas_call(
        _nms_body,
        out_shape=jax.ShapeDtypeStruct((8, 128), jnp.float32),
    )(inp)
    return out[:, :_K].T


# speedup vs baseline: 9.5944x; 1.0972x over previous
import numpy as np
import jax
import jax.numpy as jnp
from jax.experimental import pallas as pl

_N = 5000
_ROWS = 40
_NPAD = _ROWS * 128
_K = 100
_NEG = -1e30
_CLIP = float(np.float32(np.log(1333.0 / 16.0)))


def _nms_body(inp_ref, out_ref):
    px1 = inp_ref[0]
    py1 = inp_ref[1]
    px2 = inp_ref[2]
    py2 = inp_ref[3]
    tx = inp_ref[4] / 10.0
    ty = inp_ref[5] / 10.0
    tw = inp_ref[6] / 5.0
    th = inp_ref[7] / 5.0
    score = inp_ref[8]

    wa = px2 - px1
    ha = py2 - py1
    xa = (px2 + px1) * 0.5
    ya = (py2 + py1) * 0.5
    wb = jnp.exp(jnp.minimum(tw, _CLIP)) * wa
    hb = jnp.exp(jnp.minimum(th, _CLIP)) * ha
    xb = tx * wa + xa
    yb = ty * ha + ya
    x1 = jnp.clip(xb - wb * 0.5, 0.0, 1024.0)
    y1 = jnp.clip(yb - hb * 0.5, 0.0, 1024.0)
    x2 = jnp.clip(xb + wb * 0.5, 0.0, 1024.0)
    y2 = jnp.clip(yb + hb * 0.5, 0.0, 1024.0)

    idx = (jax.lax.broadcasted_iota(jnp.int32, (_ROWS, 128), 0) * 128
           + jax.lax.broadcasted_iota(jnp.int32, (_ROWS, 128), 1))
    valid0 = (score > 0.05) & ((x2 - x1) * (y2 - y1) > 0.0) & (idx < _N)
    work0 = jnp.where(valid0, score, _NEG)

    sub8 = jax.lax.broadcasted_iota(jnp.int32, (8, 128), 0)
    lane8 = jax.lax.broadcasted_iota(jnp.int32, (8, 128), 1)
    acc0 = jnp.zeros((8, 128), jnp.float32)
    z = jnp.float32(0.0)

    def gather(sel):
        return [jnp.sum(jnp.where(sel, c, z))
                for c in (x1, y1, x2, y2, inp_ref[9], inp_ref[10])] + [
                    jnp.sum(jnp.where(sel, 1.0, z))]

    def iou_row(bx1, by1, bx2, by2, bar):
        xx1 = jnp.maximum(x1, bx1)
        yy1 = jnp.maximum(y1, by1)
        xx2 = jnp.minimum(x2, bx2)
        yy2 = jnp.minimum(y2, by2)
        inter = jnp.maximum(xx2 - xx1, 0.0) * jnp.maximum(yy2 - yy1, 0.0)
        area = jnp.maximum(x2 - x1, 0.0) * jnp.maximum(y2 - y1, 0.0)
        union = area + bar - inter
        return inter / jnp.maximum(union, 1e-8)

    def row_vals(bx1, by1, bx2, by2, bco, bsi, m):
        return jnp.where(
            sub8 == 0, bx1,
            jnp.where(sub8 == 1, by1,
                      jnp.where(sub8 == 2, bx2,
                                jnp.where(sub8 == 3, by2,
                                          jnp.where(sub8 == 4, 1.0,
                                                    jnp.where(sub8 == 5, m,
                                                              jnp.where(sub8 == 6, bco, bsi)))))))

    def cond_fun(state):
        _work, _acc, cnt, alive = state
        return (cnt < _K) & alive

    def body_fun(state):
        work, acc, cnt, _alive = state
        bval = jnp.max(work)
        sel1 = work == bval
        bx1, by1, bx2, by2, bco, bsi, cnt1 = gather(sel1)
        bval2 = jnp.max(jnp.where(sel1, _NEG, work))
        v1 = (bval > _NEG * 0.5).astype(jnp.float32)

        def pair_iou(ax1, ay1, ax2, ay2, aar, ox1, oy1, ox2, oy2, oar):
            sxx1 = jnp.maximum(ax1, ox1)
            syy1 = jnp.maximum(ay1, oy1)
            sxx2 = jnp.minimum(ax2, ox2)
            syy2 = jnp.minimum(ay2, oy2)
            inter = (jnp.maximum(sxx2 - sxx1, 0.0)
                     * jnp.maximum(syy2 - syy1, 0.0))
            return inter / jnp.maximum(aar + oar - inter, 1e-8)

        def normal(_):
            bar1 = jnp.maximum(bx2 - bx1, 0.0) * jnp.maximum(by2 - by1, 0.0)
            iou1 = iou_row(bx1, by1, bx2, by2, bar1)
            w1 = jnp.where(iou1 >= 0.5, _NEG, work)
            acc1 = jnp.where(lane8 == cnt,
                             row_vals(bx1, by1, bx2, by2, bco, bsi, bval) * v1,
                             acc)
            sel2 = work == bval2
            cx1, cy1, cx2, cy2, cco, csi, cnt2 = gather(sel2)
            bval3 = jnp.max(jnp.where(sel1 | sel2, _NEG, work))
            bar2 = jnp.maximum(cx2 - cx1, 0.0) * jnp.maximum(cy2 - cy1, 0.0)
            iou12 = pair_iou(cx1, cy1, cx2, cy2, bar2,
                             bx1, by1, bx2, by2, bar1)
            keep2 = (bval2 > _NEG * 0.5) & (iou12 < 0.5) & (cnt2 < 1.5)
            iou2 = iou_row(cx1, cy1, cx2, cy2, bar2)
            w2 = jnp.where(keep2 & (iou2 >= 0.5), _NEG, w1)
            acc2 = jnp.where((lane8 == cnt + 1) & keep2,
                             row_vals(cx1, cy1, cx2, cy2, cco, csi, bval2),
                             acc1)
            sel3 = work == bval3
            dx1, dy1, dx2, dy2, dco, dsi, cnt3 = gather(sel3)
            bar3 = jnp.maximum(dx2 - dx1, 0.0) * jnp.maximum(dy2 - dy1, 0.0)
            iou13 = pair_iou(dx1, dy1, dx2, dy2, bar3,
                             bx1, by1, bx2, by2, bar1)
            iou23 = pair_iou(dx1, dy1, dx2, dy2, bar3,
                             cx1, cy1, cx2, cy2, bar2)
            keep3 = ((bval3 > _NEG * 0.5) & (iou13 < 0.5) & (cnt3 < 1.5)
                     & ((~keep2) | (iou23 < 0.5)) & (cnt2 < 1.5))
            iou3 = iou_row(dx1, dy1, dx2, dy2, bar3)
            w3 = jnp.where(keep3 & (iou3 >= 0.5), _NEG, w2)
            k2i = keep2.astype(jnp.int32)
            acc3 = jnp.where((lane8 == cnt + 1 + k2i) & keep3,
                             row_vals(dx1, dy1, dx2, dy2, dco, dsi, bval3),
                             acc2)
            return w3, acc3, cnt + 1 + k2i + keep3.astype(jnp.int32)

        def tie(_):
            fidx = (jax.lax.broadcasted_iota(jnp.int32, (_ROWS, 128), 0) * 128
                    + jax.lax.broadcasted_iota(jnp.int32, (_ROWS, 128), 1)
                    ).astype(jnp.float32)
            bfi = jnp.min(jnp.where(sel1, fidx, jnp.float32(2 ** 30)))
            selt = fidx == bfi
            tx1, ty1, tx2, ty2, tco, tsi, _c = gather(selt)
            bart = jnp.maximum(tx2 - tx1, 0.0) * jnp.maximum(ty2 - ty1, 0.0)
            iout = iou_row(tx1, ty1, tx2, ty2, bart)
            wt = jnp.where(iout >= 0.5, _NEG, work)
            acct = jnp.where(lane8 == cnt,
                             row_vals(tx1, ty1, tx2, ty2, tco, tsi, bval) * v1,
                             acc)
            return wt, acct, cnt + 1

        work_n, acc_n, cnt_n = jax.lax.cond(cnt1 > 1.5, tie, normal, None)
        alive = bval > _NEG * 0.5
        cnt_n = jnp.where(alive, cnt_n, cnt)
        return work_n, acc_n, cnt_n, alive

    state = jax.lax.while_loop(
        cond_fun, body_fun, (work0, acc0, jnp.int32(0), jnp.bool_(True)))
    out_ref[...] = state[1]


def kernel(proposals, box_logits, label_logits, box_cos_logits, box_sin_logits):
    x = jnp.concatenate([proposals, box_logits, label_logits[:, 1:2],
                         box_cos_logits[:, None], box_sin_logits[:, None]],
                        axis=1)
    x = jnp.pad(x, ((0, _NPAD - _N), (0, 0)))
    inp = x.T.reshape(11, _ROWS, 128)
    out = pl.pallas_call(
        _nms_body,
        out_shape=jax.ShapeDtypeStruct((8, 128), jnp.float32),
    )(inp)
    return out[:, :_K].T


# speedup vs baseline: 10.2615x; 1.0695x over previous
import numpy as np
import jax
import jax.numpy as jnp
from jax.experimental import pallas as pl

_N = 5000
_ROWS = 40
_NPAD = _ROWS * 128
_K = 100
_NEG = -1e30
_CLIP = float(np.float32(np.log(1333.0 / 16.0)))


def _nms_body(inp_ref, out_ref):
    px1 = inp_ref[0]
    py1 = inp_ref[1]
    px2 = inp_ref[2]
    py2 = inp_ref[3]
    tx = inp_ref[4] / 10.0
    ty = inp_ref[5] / 10.0
    tw = inp_ref[6] / 5.0
    th = inp_ref[7] / 5.0
    score = inp_ref[8]

    wa = px2 - px1
    ha = py2 - py1
    xa = (px2 + px1) * 0.5
    ya = (py2 + py1) * 0.5
    wb = jnp.exp(jnp.minimum(tw, _CLIP)) * wa
    hb = jnp.exp(jnp.minimum(th, _CLIP)) * ha
    xb = tx * wa + xa
    yb = ty * ha + ya
    x1 = jnp.clip(xb - wb * 0.5, 0.0, 1024.0)
    y1 = jnp.clip(yb - hb * 0.5, 0.0, 1024.0)
    x2 = jnp.clip(xb + wb * 0.5, 0.0, 1024.0)
    y2 = jnp.clip(yb + hb * 0.5, 0.0, 1024.0)

    idx = (jax.lax.broadcasted_iota(jnp.int32, (_ROWS, 128), 0) * 128
           + jax.lax.broadcasted_iota(jnp.int32, (_ROWS, 128), 1))
    valid0 = (score > 0.05) & ((x2 - x1) * (y2 - y1) > 0.0) & (idx < _N)
    work0 = jnp.where(valid0, score, _NEG)

    sub8 = jax.lax.broadcasted_iota(jnp.int32, (8, 128), 0)
    lane8 = jax.lax.broadcasted_iota(jnp.int32, (8, 128), 1)
    acc0 = jnp.zeros((8, 128), jnp.float32)
    z = jnp.float32(0.0)

    def gather(sel):
        return [jnp.sum(jnp.where(sel, c, z))
                for c in (x1, y1, x2, y2, inp_ref[9], inp_ref[10])] + [
                    jnp.sum(jnp.where(sel, 1.0, z))]

    def iou_row(bx1, by1, bx2, by2, bar):
        xx1 = jnp.maximum(x1, bx1)
        yy1 = jnp.maximum(y1, by1)
        xx2 = jnp.minimum(x2, bx2)
        yy2 = jnp.minimum(y2, by2)
        inter = jnp.maximum(xx2 - xx1, 0.0) * jnp.maximum(yy2 - yy1, 0.0)
        area = jnp.maximum(x2 - x1, 0.0) * jnp.maximum(y2 - y1, 0.0)
        union = area + bar - inter
        return inter / jnp.maximum(union, 1e-8)

    def row_vals(bx1, by1, bx2, by2, bco, bsi, m):
        return jnp.where(
            sub8 == 0, bx1,
            jnp.where(sub8 == 1, by1,
                      jnp.where(sub8 == 2, bx2,
                                jnp.where(sub8 == 3, by2,
                                          jnp.where(sub8 == 4, 1.0,
                                                    jnp.where(sub8 == 5, m,
                                                              jnp.where(sub8 == 6, bco, bsi)))))))

    def cond_fun(state):
        _work, _acc, cnt, alive = state
        return (cnt < _K) & alive

    def body_fun(state):
        work, acc, cnt, _alive = state
        bval = jnp.max(work)
        sel1 = work == bval
        bx1, by1, bx2, by2, bco, bsi, cnt1 = gather(sel1)
        bval2 = jnp.max(jnp.where(sel1, _NEG, work))
        v1 = (bval > _NEG * 0.5).astype(jnp.float32)

        def pair_iou(ax1, ay1, ax2, ay2, aar, ox1, oy1, ox2, oy2, oar):
            sxx1 = jnp.maximum(ax1, ox1)
            syy1 = jnp.maximum(ay1, oy1)
            sxx2 = jnp.minimum(ax2, ox2)
            syy2 = jnp.minimum(ay2, oy2)
            inter = (jnp.maximum(sxx2 - sxx1, 0.0)
                     * jnp.maximum(syy2 - syy1, 0.0))
            return inter / jnp.maximum(aar + oar - inter, 1e-8)

        def normal(_):
            bar1 = jnp.maximum(bx2 - bx1, 0.0) * jnp.maximum(by2 - by1, 0.0)
            iou1 = iou_row(bx1, by1, bx2, by2, bar1)
            w1 = jnp.where(iou1 >= 0.5, _NEG, work)
            acc1 = jnp.where(lane8 == cnt,
                             row_vals(bx1, by1, bx2, by2, bco, bsi, bval) * v1,
                             acc)
            sel2 = work == bval2
            cx1, cy1, cx2, cy2, cco, csi, cnt2 = gather(sel2)
            bval3 = jnp.max(jnp.where(sel1 | sel2, _NEG, work))
            bar2 = jnp.maximum(cx2 - cx1, 0.0) * jnp.maximum(cy2 - cy1, 0.0)
            iou12 = pair_iou(cx1, cy1, cx2, cy2, bar2,
                             bx1, by1, bx2, by2, bar1)
            keep2 = (bval2 > _NEG * 0.5) & (iou12 < 0.5) & (cnt2 < 1.5)
            iou2 = iou_row(cx1, cy1, cx2, cy2, bar2)
            w2 = jnp.where(keep2 & (iou2 >= 0.5), _NEG, w1)
            acc2 = jnp.where((lane8 == cnt + 1) & keep2,
                             row_vals(cx1, cy1, cx2, cy2, cco, csi, bval2),
                             acc1)
            sel3 = work == bval3
            dx1, dy1, dx2, dy2, dco, dsi, cnt3 = gather(sel3)
            bval4 = jnp.max(jnp.where(sel1 | sel2 | sel3, _NEG, work))
            bar3 = jnp.maximum(dx2 - dx1, 0.0) * jnp.maximum(dy2 - dy1, 0.0)
            iou13 = pair_iou(dx1, dy1, dx2, dy2, bar3,
                             bx1, by1, bx2, by2, bar1)
            iou23 = pair_iou(dx1, dy1, dx2, dy2, bar3,
                             cx1, cy1, cx2, cy2, bar2)
            keep3 = ((bval3 > _NEG * 0.5) & (iou13 < 0.5) & (cnt3 < 1.5)
                     & ((~keep2) | (iou23 < 0.5)) & (cnt2 < 1.5))
            iou3 = iou_row(dx1, dy1, dx2, dy2, bar3)
            w3 = jnp.where(keep3 & (iou3 >= 0.5), _NEG, w2)
            k2i = keep2.astype(jnp.int32)
            acc3 = jnp.where((lane8 == cnt + 1 + k2i) & keep3,
                             row_vals(dx1, dy1, dx2, dy2, dco, dsi, bval3),
                             acc2)
            sel4 = work == bval4
            ex1, ey1, ex2, ey2, eco, esi, cnt4 = gather(sel4)
            bar4 = jnp.maximum(ex2 - ex1, 0.0) * jnp.maximum(ey2 - ey1, 0.0)
            iou14 = pair_iou(ex1, ey1, ex2, ey2, bar4,
                             bx1, by1, bx2, by2, bar1)
            iou24 = pair_iou(ex1, ey1, ex2, ey2, bar4,
                             cx1, cy1, cx2, cy2, bar2)
            iou34 = pair_iou(ex1, ey1, ex2, ey2, bar4,
                             dx1, dy1, dx2, dy2, bar3)
            keep4 = ((bval4 > _NEG * 0.5) & (iou14 < 0.5) & (cnt4 < 1.5)
                     & ((~keep2) | (iou24 < 0.5))
                     & ((~keep3) | (iou34 < 0.5))
                     & (cnt2 < 1.5) & (cnt3 < 1.5))
            iou4 = iou_row(ex1, ey1, ex2, ey2, bar4)
            w4 = jnp.where(keep4 & (iou4 >= 0.5), _NEG, w3)
            k3i = keep3.astype(jnp.int32)
            acc4 = jnp.where((lane8 == cnt + 1 + k2i + k3i) & keep4,
                             row_vals(ex1, ey1, ex2, ey2, eco, esi, bval4),
                             acc3)
            return (w4, acc4,
                    cnt + 1 + k2i + k3i + keep4.astype(jnp.int32))

        def tie(_):
            fidx = (jax.lax.broadcasted_iota(jnp.int32, (_ROWS, 128), 0) * 128
                    + jax.lax.broadcasted_iota(jnp.int32, (_ROWS, 128), 1)
                    ).astype(jnp.float32)
            bfi = jnp.min(jnp.where(sel1, fidx, jnp.float32(2 ** 30)))
            selt = fidx == bfi
            tx1, ty1, tx2, ty2, tco, tsi, _c = gather(selt)
            bart = jnp.maximum(tx2 - tx1, 0.0) * jnp.maximum(ty2 - ty1, 0.0)
            iout = iou_row(tx1, ty1, tx2, ty2, bart)
            wt = jnp.where(iout >= 0.5, _NEG, work)
            acct = jnp.where(lane8 == cnt,
                             row_vals(tx1, ty1, tx2, ty2, tco, tsi, bval) * v1,
                             acc)
            return wt, acct, cnt + 1

        work_n, acc_n, cnt_n = jax.lax.cond(cnt1 > 1.5, tie, normal, None)
        alive = bval > _NEG * 0.5
        cnt_n = jnp.where(alive, cnt_n, cnt)
        return work_n, acc_n, cnt_n, alive

    state = jax.lax.while_loop(
        cond_fun, body_fun, (work0, acc0, jnp.int32(0), jnp.bool_(True)))
    out_ref[...] = state[1]


def kernel(proposals, box_logits, label_logits, box_cos_logits, box_sin_logits):
    x = jnp.concatenate([proposals, box_logits, label_logits[:, 1:2],
                         box_cos_logits[:, None], box_sin_logits[:, None]],
                        axis=1)
    x = jnp.pad(x, ((0, _NPAD - _N), (0, 0)))
    inp = x.T.reshape(11, _ROWS, 128)
    out = pl.pallas_call(
        _nms_body,
        out_shape=jax.ShapeDtypeStruct((8, 128), jnp.float32),
    )(inp)
    return out[:, :_K].T


# speedup vs baseline: 10.5810x; 1.0311x over previous
import numpy as np
import jax
import jax.numpy as jnp
from jax.experimental import pallas as pl

_N = 5000
_ROWS = 40
_NPAD = _ROWS * 128
_K = 100
_NEG = -1e30
_CLIP = float(np.float32(np.log(1333.0 / 16.0)))


def _nms_body(inp_ref, out_ref):
    px1 = inp_ref[0]
    py1 = inp_ref[1]
    px2 = inp_ref[2]
    py2 = inp_ref[3]
    tx = inp_ref[4] / 10.0
    ty = inp_ref[5] / 10.0
    tw = inp_ref[6] / 5.0
    th = inp_ref[7] / 5.0
    score = inp_ref[8]

    wa = px2 - px1
    ha = py2 - py1
    xa = (px2 + px1) * 0.5
    ya = (py2 + py1) * 0.5
    wb = jnp.exp(jnp.minimum(tw, _CLIP)) * wa
    hb = jnp.exp(jnp.minimum(th, _CLIP)) * ha
    xb = tx * wa + xa
    yb = ty * ha + ya
    x1 = jnp.clip(xb - wb * 0.5, 0.0, 1024.0)
    y1 = jnp.clip(yb - hb * 0.5, 0.0, 1024.0)
    x2 = jnp.clip(xb + wb * 0.5, 0.0, 1024.0)
    y2 = jnp.clip(yb + hb * 0.5, 0.0, 1024.0)

    idx = (jax.lax.broadcasted_iota(jnp.int32, (_ROWS, 128), 0) * 128
           + jax.lax.broadcasted_iota(jnp.int32, (_ROWS, 128), 1))
    valid0 = (score > 0.05) & ((x2 - x1) * (y2 - y1) > 0.0) & (idx < _N)
    work0 = jnp.where(valid0, score, _NEG)

    sub8 = jax.lax.broadcasted_iota(jnp.int32, (8, 128), 0)
    lane8 = jax.lax.broadcasted_iota(jnp.int32, (8, 128), 1)
    acc0 = jnp.zeros((8, 128), jnp.float32)
    z = jnp.float32(0.0)

    def gather(sel):
        return [jnp.sum(jnp.where(sel, c, z))
                for c in (x1, y1, x2, y2, inp_ref[9], inp_ref[10])] + [
                    jnp.sum(jnp.where(sel, 1.0, z))]

    def iou_row(bx1, by1, bx2, by2, bar):
        xx1 = jnp.maximum(x1, bx1)
        yy1 = jnp.maximum(y1, by1)
        xx2 = jnp.minimum(x2, bx2)
        yy2 = jnp.minimum(y2, by2)
        inter = jnp.maximum(xx2 - xx1, 0.0) * jnp.maximum(yy2 - yy1, 0.0)
        area = jnp.maximum(x2 - x1, 0.0) * jnp.maximum(y2 - y1, 0.0)
        union = area + bar - inter
        return inter / jnp.maximum(union, 1e-8)

    def row_vals(bx1, by1, bx2, by2, bco, bsi, m):
        return jnp.where(
            sub8 == 0, bx1,
            jnp.where(sub8 == 1, by1,
                      jnp.where(sub8 == 2, bx2,
                                jnp.where(sub8 == 3, by2,
                                          jnp.where(sub8 == 4, 1.0,
                                                    jnp.where(sub8 == 5, m,
                                                              jnp.where(sub8 == 6, bco, bsi)))))))

    def cond_fun(state):
        _work, _acc, cnt, alive = state
        return (cnt < _K) & alive

    def body_fun(state):
        work, acc, cnt, _alive = state
        bval = jnp.max(work)
        sel1 = work == bval
        bx1, by1, bx2, by2, bco, bsi, cnt1 = gather(sel1)
        bval2 = jnp.max(jnp.where(sel1, _NEG, work))
        v1 = (bval > _NEG * 0.5).astype(jnp.float32)

        def pair_iou(ax1, ay1, ax2, ay2, aar, ox1, oy1, ox2, oy2, oar):
            sxx1 = jnp.maximum(ax1, ox1)
            syy1 = jnp.maximum(ay1, oy1)
            sxx2 = jnp.minimum(ax2, ox2)
            syy2 = jnp.minimum(ay2, oy2)
            inter = (jnp.maximum(sxx2 - sxx1, 0.0)
                     * jnp.maximum(syy2 - syy1, 0.0))
            return inter / jnp.maximum(aar + oar - inter, 1e-8)

        def normal(_):
            bar1 = jnp.maximum(bx2 - bx1, 0.0) * jnp.maximum(by2 - by1, 0.0)
            iou1 = iou_row(bx1, by1, bx2, by2, bar1)
            w1 = jnp.where(iou1 >= 0.5, _NEG, work)
            acc1 = jnp.where(lane8 == cnt,
                             row_vals(bx1, by1, bx2, by2, bco, bsi, bval) * v1,
                             acc)
            sel2 = work == bval2
            cx1, cy1, cx2, cy2, cco, csi, cnt2 = gather(sel2)
            bval3 = jnp.max(jnp.where(sel1 | sel2, _NEG, work))
            bar2 = jnp.maximum(cx2 - cx1, 0.0) * jnp.maximum(cy2 - cy1, 0.0)
            iou12 = pair_iou(cx1, cy1, cx2, cy2, bar2,
                             bx1, by1, bx2, by2, bar1)
            keep2 = (bval2 > _NEG * 0.5) & (iou12 < 0.5) & (cnt2 < 1.5)
            iou2 = iou_row(cx1, cy1, cx2, cy2, bar2)
            w2 = jnp.where(keep2 & (iou2 >= 0.5), _NEG, w1)
            acc2 = jnp.where((lane8 == cnt + 1) & keep2,
                             row_vals(cx1, cy1, cx2, cy2, cco, csi, bval2),
                             acc1)
            sel3 = work == bval3
            dx1, dy1, dx2, dy2, dco, dsi, cnt3 = gather(sel3)
            bval4 = jnp.max(jnp.where(sel1 | sel2 | sel3, _NEG, work))
            bar3 = jnp.maximum(dx2 - dx1, 0.0) * jnp.maximum(dy2 - dy1, 0.0)
            iou13 = pair_iou(dx1, dy1, dx2, dy2, bar3,
                             bx1, by1, bx2, by2, bar1)
            iou23 = pair_iou(dx1, dy1, dx2, dy2, bar3,
                             cx1, cy1, cx2, cy2, bar2)
            keep3 = ((bval3 > _NEG * 0.5) & (iou13 < 0.5) & (cnt3 < 1.5)
                     & ((~keep2) | (iou23 < 0.5)) & (cnt2 < 1.5))
            iou3 = iou_row(dx1, dy1, dx2, dy2, bar3)
            w3 = jnp.where(keep3 & (iou3 >= 0.5), _NEG, w2)
            k2i = keep2.astype(jnp.int32)
            acc3 = jnp.where((lane8 == cnt + 1 + k2i) & keep3,
                             row_vals(dx1, dy1, dx2, dy2, dco, dsi, bval3),
                             acc2)
            sel4 = work == bval4
            ex1, ey1, ex2, ey2, eco, esi, cnt4 = gather(sel4)
            bval5 = jnp.max(jnp.where(sel1 | sel2 | sel3 | sel4, _NEG, work))
            bar4 = jnp.maximum(ex2 - ex1, 0.0) * jnp.maximum(ey2 - ey1, 0.0)
            iou14 = pair_iou(ex1, ey1, ex2, ey2, bar4,
                             bx1, by1, bx2, by2, bar1)
            iou24 = pair_iou(ex1, ey1, ex2, ey2, bar4,
                             cx1, cy1, cx2, cy2, bar2)
            iou34 = pair_iou(ex1, ey1, ex2, ey2, bar4,
                             dx1, dy1, dx2, dy2, bar3)
            keep4 = ((bval4 > _NEG * 0.5) & (iou14 < 0.5) & (cnt4 < 1.5)
                     & ((~keep2) | (iou24 < 0.5))
                     & ((~keep3) | (iou34 < 0.5))
                     & (cnt2 < 1.5) & (cnt3 < 1.5))
            iou4 = iou_row(ex1, ey1, ex2, ey2, bar4)
            w4 = jnp.where(keep4 & (iou4 >= 0.5), _NEG, w3)
            k3i = keep3.astype(jnp.int32)
            acc4 = jnp.where((lane8 == cnt + 1 + k2i + k3i) & keep4,
                             row_vals(ex1, ey1, ex2, ey2, eco, esi, bval4),
                             acc3)
            sel5 = work == bval5
            fx1, fy1, fx2, fy2, fco, fsi, cnt5 = gather(sel5)
            bar5 = jnp.maximum(fx2 - fx1, 0.0) * jnp.maximum(fy2 - fy1, 0.0)
            iou15 = pair_iou(fx1, fy1, fx2, fy2, bar5,
                             bx1, by1, bx2, by2, bar1)
            iou25 = pair_iou(fx1, fy1, fx2, fy2, bar5,
                             cx1, cy1, cx2, cy2, bar2)
            iou35 = pair_iou(fx1, fy1, fx2, fy2, bar5,
                             dx1, dy1, dx2, dy2, bar3)
            iou45 = pair_iou(fx1, fy1, fx2, fy2, bar5,
                             ex1, ey1, ex2, ey2, bar4)
            keep5 = ((bval5 > _NEG * 0.5) & (iou15 < 0.5) & (cnt5 < 1.5)
                     & ((~keep2) | (iou25 < 0.5))
                     & ((~keep3) | (iou35 < 0.5))
                     & ((~keep4) | (iou45 < 0.5))
                     & (cnt2 < 1.5) & (cnt3 < 1.5) & (cnt4 < 1.5))
            iou5 = iou_row(fx1, fy1, fx2, fy2, bar5)
            w5 = jnp.where(keep5 & (iou5 >= 0.5), _NEG, w4)
            k4i = keep4.astype(jnp.int32)
            acc5 = jnp.where((lane8 == cnt + 1 + k2i + k3i + k4i) & keep5,
                             row_vals(fx1, fy1, fx2, fy2, fco, fsi, bval5),
                             acc4)
            return (w5, acc5,
                    cnt + 1 + k2i + k3i + k4i + keep5.astype(jnp.int32))

        def tie(_):
            fidx = (jax.lax.broadcasted_iota(jnp.int32, (_ROWS, 128), 0) * 128
                    + jax.lax.broadcasted_iota(jnp.int32, (_ROWS, 128), 1)
                    ).astype(jnp.float32)
            bfi = jnp.min(jnp.where(sel1, fidx, jnp.float32(2 ** 30)))
            selt = fidx == bfi
            tx1, ty1, tx2, ty2, tco, tsi, _c = gather(selt)
            bart = jnp.maximum(tx2 - tx1, 0.0) * jnp.maximum(ty2 - ty1, 0.0)
            iout = iou_row(tx1, ty1, tx2, ty2, bart)
            wt = jnp.where(iout >= 0.5, _NEG, work)
            acct = jnp.where(lane8 == cnt,
                             row_vals(tx1, ty1, tx2, ty2, tco, tsi, bval) * v1,
                             acc)
            return wt, acct, cnt + 1

        work_n, acc_n, cnt_n = jax.lax.cond(cnt1 > 1.5, tie, normal, None)
        alive = bval > _NEG * 0.5
        cnt_n = jnp.where(alive, cnt_n, cnt)
        return work_n, acc_n, cnt_n, alive

    state = jax.lax.while_loop(
        cond_fun, body_fun, (work0, acc0, jnp.int32(0), jnp.bool_(True)))
    out_ref[...] = state[1]


def kernel(proposals, box_logits, label_logits, box_cos_logits, box_sin_logits):
    x = jnp.concatenate([proposals, box_logits, label_logits[:, 1:2],
                         box_cos_logits[:, None], box_sin_logits[:, None]],
                        axis=1)
    x = jnp.pad(x, ((0, _NPAD - _N), (0, 0)))
    inp = x.T.reshape(11, _ROWS, 128)
    out = pl.pallas_call(
        _nms_body,
        out_shape=jax.ShapeDtypeStruct((8, 128), jnp.float32),
    )(inp)
    return out[:, :_K].T
